# baseline (device time: 425371 ns/iter reference)
import jax
import jax.numpy as jnp
from jax import lax
from jax.experimental import pallas as pl
from jax.experimental.pallas import tpu as pltpu

N_DEV = 32


def kernel(x, w_mat, scale_x, scale_w):
    k_global, _ = x.shape
    _, n = w_mat.shape
    m_per = k_global // N_DEV

    def body(x_ref, w_ref, sx_ref, sw_ref, out_ref,
             send_buf, recv_buf, send_sems, recv_sems, credit_sem):
        d = lax.axis_index("i")
        left = lax.rem(d - 1 + N_DEV, N_DEV)
        right = lax.rem(d + 1, N_DEV)

        barrier_sem = pltpu.get_barrier_semaphore()
        for nbr in (left, right):
            pl.semaphore_signal(barrier_sem, inc=1, device_id=(nbr,),
                                device_id_type=pl.DeviceIdType.MESH)
        pl.semaphore_wait(barrier_sem, 2)

        w = w_ref[:, :].astype(jnp.bfloat16)
        scale = sx_ref[0] * sw_ref[0]

        for s in range(N_DEV):
            c = lax.rem(d - 1 - s + 2 * N_DEV, N_DEV)
            xs = x_ref[pl.ds(c * m_per, m_per), :].astype(jnp.bfloat16)
            p = lax.dot_general(xs, w, (((1,), (0,)), ((), ())),
                                preferred_element_type=jnp.float32)
            if s == 0:
                acc = p
            else:
                acc = p + recv_buf[(s - 1) % 2, :, :]
                if s <= N_DEV - 3:
                    pl.semaphore_signal(credit_sem, inc=1,
                                        device_id=(left,),
                                        device_id_type=pl.DeviceIdType.MESH)
            if s < N_DEV - 1:
                send_buf[s % 2, :, :] = acc
                if s >= 2:
                    pl.semaphore_wait(credit_sem, 1)
                rdma = pltpu.make_async_remote_copy(
                    src_ref=send_buf.at[s % 2],
                    dst_ref=recv_buf.at[s % 2],
                    send_sem=send_sems.at[s % 2],
                    recv_sem=recv_sems.at[s % 2],
                    device_id=(right,),
                    device_id_type=pl.DeviceIdType.MESH,
                )
                rdma.start()
                rdma.wait()
            else:
                out_ref[:, :] = acc * scale

    return pl.pallas_call(
        body,
        out_shape=jax.ShapeDtypeStruct((m_per, n), jnp.float32),
        in_specs=[pl.BlockSpec(memory_space=pltpu.VMEM)] * 4,
        out_specs=pl.BlockSpec(memory_space=pltpu.VMEM),
        scratch_shapes=[
            pltpu.VMEM((2, m_per, n), jnp.float32),
            pltpu.VMEM((2, m_per, n), jnp.float32),
            pltpu.SemaphoreType.DMA((2,)),
            pltpu.SemaphoreType.DMA((2,)),
            pltpu.SemaphoreType.REGULAR,
        ],
        compiler_params=pltpu.CompilerParams(collective_id=0),
    )(x, w_mat, scale_x, scale_w)


# device time: 415989 ns/iter; 1.0226x vs baseline; 1.0226x over previous
import jax
import jax.numpy as jnp
from jax import lax
from jax.experimental import pallas as pl
from jax.experimental.pallas import tpu as pltpu

N_DEV = 32
WIRE_DTYPE = jnp.float32


def kernel(x, w_mat, scale_x, scale_w):
    k_global, _ = x.shape
    _, n = w_mat.shape
    m_per = k_global // N_DEV
    nh = n // 2

    def body(x_ref, w_ref, sx_ref, sw_ref, out_ref,
             send_cw, recv_cw, send_ccw, recv_ccw,
             send_sems_cw, recv_sems_cw, send_sems_ccw, recv_sems_ccw,
             credit_cw, credit_ccw):
        d = lax.axis_index("i")
        left = lax.rem(d - 1 + N_DEV, N_DEV)
        right = lax.rem(d + 1, N_DEV)

        barrier_sem = pltpu.get_barrier_semaphore()
        for nbr in (left, right):
            pl.semaphore_signal(barrier_sem, inc=1, device_id=(nbr,),
                                device_id_type=pl.DeviceIdType.MESH)
        pl.semaphore_wait(barrier_sem, 2)

        wl = w_ref[:, :nh].astype(jnp.bfloat16)
        wr = w_ref[:, nh:].astype(jnp.bfloat16)
        scale = sx_ref[0] * sw_ref[0]

        def partials(s):
            c1 = lax.rem(d - 1 - s + 2 * N_DEV, N_DEV)
            c2 = lax.rem(d + 1 + s, N_DEV)
            x1 = x_ref[pl.ds(c1 * m_per, m_per), :].astype(jnp.bfloat16)
            x2 = x_ref[pl.ds(c2 * m_per, m_per), :].astype(jnp.bfloat16)
            dims = (((1,), (0,)), ((), ()))
            p1 = lax.dot_general(x1, wl, dims,
                                 preferred_element_type=jnp.float32)
            p2 = lax.dot_general(x2, wr, dims,
                                 preferred_element_type=jnp.float32)
            return p1, p2

        p1, p2 = partials(0)
        for s in range(N_DEV):
            if s == 0:
                acc1, acc2 = p1, p2
            else:
                acc1 = p1 + recv_cw[(s - 1) % 2, :, :].astype(jnp.float32)
                acc2 = p2 + recv_ccw[(s - 1) % 2, :, :].astype(jnp.float32)
                if s <= N_DEV - 3:
                    pl.semaphore_signal(credit_cw, inc=1,
                                        device_id=(left,),
                                        device_id_type=pl.DeviceIdType.MESH)
                    pl.semaphore_signal(credit_ccw, inc=1,
                                        device_id=(right,),
                                        device_id_type=pl.DeviceIdType.MESH)
            if s < N_DEV - 1:
                send_cw[s % 2, :, :] = acc1.astype(WIRE_DTYPE)
                send_ccw[s % 2, :, :] = acc2.astype(WIRE_DTYPE)
                if s >= 2:
                    pl.semaphore_wait(credit_cw, 1)
                    pl.semaphore_wait(credit_ccw, 1)
                rdma1 = pltpu.make_async_remote_copy(
                    src_ref=send_cw.at[s % 2],
                    dst_ref=recv_cw.at[s % 2],
                    send_sem=send_sems_cw.at[s % 2],
                    recv_sem=recv_sems_cw.at[s % 2],
                    device_id=(right,),
                    device_id_type=pl.DeviceIdType.MESH,
                )
                rdma2 = pltpu.make_async_remote_copy(
                    src_ref=send_ccw.at[s % 2],
                    dst_ref=recv_ccw.at[s % 2],
                    send_sem=send_sems_ccw.at[s % 2],
                    recv_sem=recv_sems_ccw.at[s % 2],
                    device_id=(left,),
                    device_id_type=pl.DeviceIdType.MESH,
                )
                rdma1.start()
                rdma2.start()
                p1, p2 = partials(s + 1)
                rdma1.wait()
                rdma2.wait()
            else:
                out_ref[:, :nh] = acc1 * scale
                out_ref[:, nh:] = acc2 * scale

    return pl.pallas_call(
        body,
        out_shape=jax.ShapeDtypeStruct((m_per, n), jnp.float32),
        in_specs=[pl.BlockSpec(memory_space=pltpu.VMEM)] * 4,
        out_specs=pl.BlockSpec(memory_space=pltpu.VMEM),
        scratch_shapes=[
            pltpu.VMEM((2, m_per, nh), WIRE_DTYPE),
            pltpu.VMEM((2, m_per, nh), WIRE_DTYPE),
            pltpu.VMEM((2, m_per, nh), WIRE_DTYPE),
            pltpu.VMEM((2, m_per, nh), WIRE_DTYPE),
            pltpu.SemaphoreType.DMA((2,)),
            pltpu.SemaphoreType.DMA((2,)),
            pltpu.SemaphoreType.DMA((2,)),
            pltpu.SemaphoreType.DMA((2,)),
            pltpu.SemaphoreType.REGULAR,
            pltpu.SemaphoreType.REGULAR,
        ],
        compiler_params=pltpu.CompilerParams(collective_id=0),
    )(x, w_mat, scale_x, scale_w)


# device time: 235340 ns/iter; 1.8075x vs baseline; 1.7676x over previous
import jax
import jax.numpy as jnp
from jax import lax
from jax.experimental import pallas as pl
from jax.experimental.pallas import tpu as pltpu

N_DEV = 32
WIRE_DTYPE = jnp.bfloat16


def kernel(x, w_mat, scale_x, scale_w):
    k_global, _ = x.shape
    _, n = w_mat.shape
    m_per = k_global // N_DEV
    nh = n // 2

    def body(x_ref, w_ref, sx_ref, sw_ref, out_ref,
             send_cw, recv_cw, send_ccw, recv_ccw,
             send_sems_cw, recv_sems_cw, send_sems_ccw, recv_sems_ccw,
             credit_cw, credit_ccw):
        d = lax.axis_index("i")
        left = lax.rem(d - 1 + N_DEV, N_DEV)
        right = lax.rem(d + 1, N_DEV)

        barrier_sem = pltpu.get_barrier_semaphore()
        for nbr in (left, right):
            pl.semaphore_signal(barrier_sem, inc=1, device_id=(nbr,),
                                device_id_type=pl.DeviceIdType.MESH)
        pl.semaphore_wait(barrier_sem, 2)

        wl = w_ref[:, :nh].astype(jnp.bfloat16)
        wr = w_ref[:, nh:].astype(jnp.bfloat16)
        scale = sx_ref[0] * sw_ref[0]

        def partials(s):
            c1 = lax.rem(d - 1 - s + 2 * N_DEV, N_DEV)
            c2 = lax.rem(d + 1 + s, N_DEV)
            x1 = x_ref[pl.ds(c1 * m_per, m_per), :].astype(jnp.bfloat16)
            x2 = x_ref[pl.ds(c2 * m_per, m_per), :].astype(jnp.bfloat16)
            dims = (((1,), (0,)), ((), ()))
            p1 = lax.dot_general(x1, wl, dims,
                                 preferred_element_type=jnp.float32)
            p2 = lax.dot_general(x2, wr, dims,
                                 preferred_element_type=jnp.float32)
            return p1, p2

        p1, p2 = partials(0)
        for s in range(N_DEV):
            if s == 0:
                acc1, acc2 = p1, p2
            else:
                acc1 = p1 + recv_cw[(s - 1) % 2, :, :].astype(jnp.float32)
                acc2 = p2 + recv_ccw[(s - 1) % 2, :, :].astype(jnp.float32)
                if s <= N_DEV - 3:
                    pl.semaphore_signal(credit_cw, inc=1,
                                        device_id=(left,),
                                        device_id_type=pl.DeviceIdType.MESH)
                    pl.semaphore_signal(credit_ccw, inc=1,
                                        device_id=(right,),
                                        device_id_type=pl.DeviceIdType.MESH)
            if s < N_DEV - 1:
                send_cw[s % 2, :, :] = acc1.astype(WIRE_DTYPE)
                send_ccw[s % 2, :, :] = acc2.astype(WIRE_DTYPE)
                if s >= 2:
                    pl.semaphore_wait(credit_cw, 1)
                    pl.semaphore_wait(credit_ccw, 1)
                rdma1 = pltpu.make_async_remote_copy(
                    src_ref=send_cw.at[s % 2],
                    dst_ref=recv_cw.at[s % 2],
                    send_sem=send_sems_cw.at[s % 2],
                    recv_sem=recv_sems_cw.at[s % 2],
                    device_id=(right,),
                    device_id_type=pl.DeviceIdType.MESH,
                )
                rdma2 = pltpu.make_async_remote_copy(
                    src_ref=send_ccw.at[s % 2],
                    dst_ref=recv_ccw.at[s % 2],
                    send_sem=send_sems_ccw.at[s % 2],
                    recv_sem=recv_sems_ccw.at[s % 2],
                    device_id=(left,),
                    device_id_type=pl.DeviceIdType.MESH,
                )
                rdma1.start()
                rdma2.start()
                p1, p2 = partials(s + 1)
                rdma1.wait()
                rdma2.wait()
            else:
                out_ref[:, :nh] = acc1 * scale
                out_ref[:, nh:] = acc2 * scale

    return pl.pallas_call(
        body,
        out_shape=jax.ShapeDtypeStruct((m_per, n), jnp.float32),
        in_specs=[pl.BlockSpec(memory_space=pltpu.VMEM)] * 4,
        out_specs=pl.BlockSpec(memory_space=pltpu.VMEM),
        scratch_shapes=[
            pltpu.VMEM((2, m_per, nh), WIRE_DTYPE),
            pltpu.VMEM((2, m_per, nh), WIRE_DTYPE),
            pltpu.VMEM((2, m_per, nh), WIRE_DTYPE),
            pltpu.VMEM((2, m_per, nh), WIRE_DTYPE),
            pltpu.SemaphoreType.DMA((2,)),
            pltpu.SemaphoreType.DMA((2,)),
            pltpu.SemaphoreType.DMA((2,)),
            pltpu.SemaphoreType.DMA((2,)),
            pltpu.SemaphoreType.REGULAR,
            pltpu.SemaphoreType.REGULAR,
        ],
        compiler_params=pltpu.CompilerParams(collective_id=0),
    )(x, w_mat, scale_x, scale_w)


# device time: 170635 ns/iter; 2.4929x vs baseline; 1.3792x over previous
import jax
import jax.numpy as jnp
import numpy as np
from jax import lax
from jax.experimental import pallas as pl
from jax.experimental.pallas import tpu as pltpu

N_DEV = 32
N_SEG = 1
WIRE_DTYPE = jnp.bfloat16

_XY_TO_P = {(0, 0): 0, (1, 0): 1, (1, 1): 2, (0, 1): 3,
            (0, 2): 4, (1, 2): 5, (1, 3): 6, (0, 3): 7}


def _build_cycle() -> np.ndarray:
    path = []
    for y in range(4):
        zs = range(4) if y % 2 == 0 else range(3, -1, -1)
        path.extend((y, z) for z in zs)
    cycle = [(0, y, z) for (y, z) in path]
    cycle += [(1, y, z) for (y, z) in reversed(path)]
    return np.array([z * 8 + _XY_TO_P[(x, y)] for (x, y, z) in cycle],
                    dtype=np.int32)


PERM = _build_cycle()
PERM_INV = np.zeros(N_DEV, np.int32)
PERM_INV[PERM] = np.arange(N_DEV, dtype=np.int32)


def kernel(x, w_mat, scale_x, scale_w):
    k_global, _ = x.shape
    _, n = w_mat.shape
    m_per = k_global // N_DEV
    nh = n // 2
    nq = nh // N_SEG
    n_lanes = 2 * N_SEG

    perm = jnp.asarray(PERM)
    pos = jnp.asarray(PERM_INV)[lax.axis_index("i")]
    left = perm[(pos - 1) % N_DEV].reshape(1)
    right = perm[(pos + 1) % N_DEV].reshape(1)
    sidx = jnp.arange(N_DEV, dtype=jnp.int32)
    cs_cw = perm[(pos - 1 - sidx) % N_DEV]
    cs_ccw = perm[(pos + 1 + sidx) % N_DEV]

    def body(x_ref, w_ref, sx_ref, sw_ref, left_ref, right_ref,
             cs_cw_ref, cs_ccw_ref, out_ref, *scratch):
        lanes = [scratch[5 * li:5 * li + 5] for li in range(n_lanes)]

        lft = left_ref[0]
        rgt = right_ref[0]
        dst = [rgt] * N_SEG + [lft] * N_SEG
        ups = [lft] * N_SEG + [rgt] * N_SEG

        barrier_sem = pltpu.get_barrier_semaphore()
        for nbr in (lft, rgt):
            pl.semaphore_signal(barrier_sem, inc=1, device_id=(nbr,),
                                device_id_type=pl.DeviceIdType.MESH)
        pl.semaphore_wait(barrier_sem, 2)

        wl = w_ref[:, :nh].astype(jnp.bfloat16)
        wr = w_ref[:, nh:].astype(jnp.bfloat16)
        scale = sx_ref[0] * sw_ref[0]

        def partials(s):
            c1 = cs_cw_ref[s]
            c2 = cs_ccw_ref[s]
            x1 = x_ref[pl.ds(c1 * m_per, m_per), :].astype(jnp.bfloat16)
            x2 = x_ref[pl.ds(c2 * m_per, m_per), :].astype(jnp.bfloat16)
            dims = (((1,), (0,)), ((), ()))
            p_cw = lax.dot_general(x1, wl, dims,
                                   preferred_element_type=jnp.float32)
            p_ccw = lax.dot_general(x2, wr, dims,
                                    preferred_element_type=jnp.float32)
            ps = [p_cw[:, g * nq:(g + 1) * nq] for g in range(N_SEG)]
            ps += [p_ccw[:, g * nq:(g + 1) * nq] for g in range(N_SEG)]
            return ps

        ps = partials(0)
        for s in range(N_DEV):
            accs = []
            for li in range(n_lanes):
                _, recv_buf, _, _, credit = lanes[li]
                if s == 0:
                    acc = ps[li]
                else:
                    acc = ps[li] + recv_buf[(s - 1) % 2, :, :].astype(
                        jnp.float32)
                    if s <= N_DEV - 3:
                        pl.semaphore_signal(
                            credit, inc=1, device_id=(ups[li],),
                            device_id_type=pl.DeviceIdType.MESH)
                accs.append(acc)
            if s < N_DEV - 1:
                rdmas = []
                for li in range(n_lanes):
                    send_buf, recv_buf, send_sems, recv_sems, credit = \
                        lanes[li]
                    send_buf[s % 2, :, :] = accs[li].astype(WIRE_DTYPE)
                    if s >= 2:
                        pl.semaphore_wait(credit, 1)
                    rdma = pltpu.make_async_remote_copy(
                        src_ref=send_buf.at[s % 2],
                        dst_ref=recv_buf.at[s % 2],
                        send_sem=send_sems.at[s % 2],
                        recv_sem=recv_sems.at[s % 2],
                        device_id=(dst[li],),
                        device_id_type=pl.DeviceIdType.MESH,
                    )
                    rdma.start()
                    rdmas.append(rdma)
                ps = partials(s + 1)
                for rdma in rdmas:
                    rdma.wait()
            else:
                for li in range(n_lanes):
                    out_ref[:, li * nq:(li + 1) * nq] = accs[li] * scale

    lane_scratch = []
    for _ in range(n_lanes):
        lane_scratch += [
            pltpu.VMEM((2, m_per, nq), WIRE_DTYPE),
            pltpu.VMEM((2, m_per, nq), WIRE_DTYPE),
            pltpu.SemaphoreType.DMA((2,)),
            pltpu.SemaphoreType.DMA((2,)),
            pltpu.SemaphoreType.REGULAR,
        ]

    return pl.pallas_call(
        body,
        out_shape=jax.ShapeDtypeStruct((m_per, n), jnp.float32),
        in_specs=[pl.BlockSpec(memory_space=pltpu.VMEM)] * 4
        + [pl.BlockSpec(memory_space=pltpu.SMEM)] * 4,
        out_specs=pl.BlockSpec(memory_space=pltpu.VMEM),
        scratch_shapes=lane_scratch,
        compiler_params=pltpu.CompilerParams(collective_id=0),
    )(x, w_mat, scale_x, scale_w, left, right, cs_cw, cs_ccw)


# device time: 159896 ns/iter; 2.6603x vs baseline; 1.0672x over previous
import jax
import jax.numpy as jnp
import numpy as np
from jax import lax
from jax.experimental import pallas as pl
from jax.experimental.pallas import tpu as pltpu

N_DEV = 32
N_SEG = 2
WIRE_DTYPE = jnp.bfloat16

_XY_TO_P = {(0, 0): 0, (1, 0): 1, (1, 1): 2, (0, 1): 3,
            (0, 2): 4, (1, 2): 5, (1, 3): 6, (0, 3): 7}


def _build_cycle() -> np.ndarray:
    path = []
    for y in range(4):
        zs = range(4) if y % 2 == 0 else range(3, -1, -1)
        path.extend((y, z) for z in zs)
    cycle = [(0, y, z) for (y, z) in path]
    cycle += [(1, y, z) for (y, z) in reversed(path)]
    return np.array([z * 8 + _XY_TO_P[(x, y)] for (x, y, z) in cycle],
                    dtype=np.int32)


PERM = _build_cycle()
PERM_INV = np.zeros(N_DEV, np.int32)
PERM_INV[PERM] = np.arange(N_DEV, dtype=np.int32)


def kernel(x, w_mat, scale_x, scale_w):
    k_global, _ = x.shape
    _, n = w_mat.shape
    m_per = k_global // N_DEV
    nh = n // 2
    nq = nh // N_SEG
    n_lanes = 2 * N_SEG

    perm = jnp.asarray(PERM)
    pos = jnp.asarray(PERM_INV)[lax.axis_index("i")]
    left = perm[(pos - 1) % N_DEV].reshape(1)
    right = perm[(pos + 1) % N_DEV].reshape(1)
    sidx = jnp.arange(N_DEV, dtype=jnp.int32)
    cs_cw = perm[(pos - 1 - sidx) % N_DEV]
    cs_ccw = perm[(pos + 1 + sidx) % N_DEV]

    def body(x_ref, w_ref, sx_ref, sw_ref, left_ref, right_ref,
             cs_cw_ref, cs_ccw_ref, out_ref, *scratch):
        lanes = [scratch[5 * li:5 * li + 5] for li in range(n_lanes)]

        lft = left_ref[0]
        rgt = right_ref[0]
        dst = [rgt] * N_SEG + [lft] * N_SEG
        ups = [lft] * N_SEG + [rgt] * N_SEG

        barrier_sem = pltpu.get_barrier_semaphore()
        for nbr in (lft, rgt):
            pl.semaphore_signal(barrier_sem, inc=1, device_id=(nbr,),
                                device_id_type=pl.DeviceIdType.MESH)
        pl.semaphore_wait(barrier_sem, 2)

        wl = w_ref[:, :nh].astype(jnp.bfloat16)
        wr = w_ref[:, nh:].astype(jnp.bfloat16)
        scale = sx_ref[0] * sw_ref[0]

        def partials(s):
            c1 = cs_cw_ref[s]
            c2 = cs_ccw_ref[s]
            x1 = x_ref[pl.ds(c1 * m_per, m_per), :].astype(jnp.bfloat16)
            x2 = x_ref[pl.ds(c2 * m_per, m_per), :].astype(jnp.bfloat16)
            dims = (((1,), (0,)), ((), ()))
            p_cw = lax.dot_general(x1, wl, dims,
                                   preferred_element_type=jnp.float32)
            p_ccw = lax.dot_general(x2, wr, dims,
                                    preferred_element_type=jnp.float32)
            ps = [p_cw[:, g * nq:(g + 1) * nq] for g in range(N_SEG)]
            ps += [p_ccw[:, g * nq:(g + 1) * nq] for g in range(N_SEG)]
            return ps

        ps = partials(0)
        for s in range(N_DEV):
            accs = []
            for li in range(n_lanes):
                _, recv_buf, _, _, credit = lanes[li]
                if s == 0:
                    acc = ps[li]
                else:
                    acc = ps[li] + recv_buf[(s - 1) % 2, :, :].astype(
                        jnp.float32)
                    if s <= N_DEV - 3:
                        pl.semaphore_signal(
                            credit, inc=1, device_id=(ups[li],),
                            device_id_type=pl.DeviceIdType.MESH)
                accs.append(acc)
            if s < N_DEV - 1:
                rdmas = []
                for li in range(n_lanes):
                    send_buf, recv_buf, send_sems, recv_sems, credit = \
                        lanes[li]
                    send_buf[s % 2, :, :] = accs[li].astype(WIRE_DTYPE)
                    if s >= 2:
                        pl.semaphore_wait(credit, 1)
                    rdma = pltpu.make_async_remote_copy(
                        src_ref=send_buf.at[s % 2],
                        dst_ref=recv_buf.at[s % 2],
                        send_sem=send_sems.at[s % 2],
                        recv_sem=recv_sems.at[s % 2],
                        device_id=(dst[li],),
                        device_id_type=pl.DeviceIdType.MESH,
                    )
                    rdma.start()
                    rdmas.append(rdma)
                ps = partials(s + 1)
                for rdma in rdmas:
                    rdma.wait()
            else:
                for li in range(n_lanes):
                    out_ref[:, li * nq:(li + 1) * nq] = accs[li] * scale

    lane_scratch = []
    for _ in range(n_lanes):
        lane_scratch += [
            pltpu.VMEM((2, m_per, nq), WIRE_DTYPE),
            pltpu.VMEM((2, m_per, nq), WIRE_DTYPE),
            pltpu.SemaphoreType.DMA((2,)),
            pltpu.SemaphoreType.DMA((2,)),
            pltpu.SemaphoreType.REGULAR,
        ]

    return pl.pallas_call(
        body,
        out_shape=jax.ShapeDtypeStruct((m_per, n), jnp.float32),
        in_specs=[pl.BlockSpec(memory_space=pltpu.VMEM)] * 4
        + [pl.BlockSpec(memory_space=pltpu.SMEM)] * 4,
        out_specs=pl.BlockSpec(memory_space=pltpu.VMEM),
        scratch_shapes=lane_scratch,
        compiler_params=pltpu.CompilerParams(collective_id=0),
    )(x, w_mat, scale_x, scale_w, left, right, cs_cw, cs_ccw)


# device time: 116425 ns/iter; 3.6536x vs baseline; 1.3734x over previous
import jax
import jax.numpy as jnp
import numpy as np
from jax import lax
from jax.experimental import pallas as pl
from jax.experimental.pallas import tpu as pltpu

N_DEV = 32
N_SEG = 2
WIRE_DTYPE = jnp.bfloat16

_XY_TO_P = {(0, 0): 0, (1, 0): 1, (1, 1): 2, (0, 1): 3,
            (0, 2): 4, (1, 2): 5, (1, 3): 6, (0, 3): 7}


def _build_cycle() -> np.ndarray:
    path = []
    for y in range(4):
        zs = range(4) if y % 2 == 0 else range(3, -1, -1)
        path.extend((y, z) for z in zs)
    cycle = [(0, y, z) for (y, z) in path]
    cycle += [(1, y, z) for (y, z) in reversed(path)]
    return np.array([z * 8 + _XY_TO_P[(x, y)] for (x, y, z) in cycle],
                    dtype=np.int32)


PERM = _build_cycle()
PERM_INV = np.zeros(N_DEV, np.int32)
PERM_INV[PERM] = np.arange(N_DEV, dtype=np.int32)


def kernel(x, w_mat, scale_x, scale_w):
    k_global, _ = x.shape
    _, n = w_mat.shape
    m_per = k_global // N_DEV
    nh = n // 2
    nq = nh // N_SEG
    n_lanes = 2 * N_SEG

    perm = jnp.asarray(PERM)
    pos = jnp.asarray(PERM_INV)[lax.axis_index("i")]
    left = perm[(pos - 1) % N_DEV].reshape(1)
    right = perm[(pos + 1) % N_DEV].reshape(1)
    sidx = jnp.arange(N_DEV, dtype=jnp.int32)
    cs_cw = perm[(pos - 1 - sidx) % N_DEV]
    cs_ccw = perm[(pos + 1 + sidx) % N_DEV]

    def body(x_ref, w_ref, sx_ref, sw_ref, left_ref, right_ref,
             cs_cw_ref, cs_ccw_ref, out_ref, *scratch):
        lanes = [scratch[5 * li:5 * li + 5] for li in range(n_lanes)]

        lft = left_ref[0]
        rgt = right_ref[0]
        dst = [rgt] * N_SEG + [lft] * N_SEG
        ups = [lft] * N_SEG + [rgt] * N_SEG

        barrier_sem = pltpu.get_barrier_semaphore()
        for nbr in (lft, rgt):
            pl.semaphore_signal(barrier_sem, inc=1, device_id=(nbr,),
                                device_id_type=pl.DeviceIdType.MESH)
        pl.semaphore_wait(barrier_sem, 2)

        wl = w_ref[:, :nh].astype(jnp.bfloat16)
        wr = w_ref[:, nh:].astype(jnp.bfloat16)
        scale = sx_ref[0] * sw_ref[0]

        def partials(s):
            c1 = cs_cw_ref[s]
            c2 = cs_ccw_ref[s]
            x1 = x_ref[pl.ds(c1 * m_per, m_per), :].astype(jnp.bfloat16)
            x2 = x_ref[pl.ds(c2 * m_per, m_per), :].astype(jnp.bfloat16)
            dims = (((1,), (0,)), ((), ()))
            p_cw = lax.dot_general(x1, wl, dims,
                                   preferred_element_type=jnp.float32)
            p_ccw = lax.dot_general(x2, wr, dims,
                                    preferred_element_type=jnp.float32)
            ps = [p_cw[:, g * nq:(g + 1) * nq] for g in range(N_SEG)]
            ps += [p_ccw[:, g * nq:(g + 1) * nq] for g in range(N_SEG)]
            return ps

        ps = partials(0)
        prev = [None] * n_lanes
        prev2 = [None] * n_lanes
        for s in range(N_DEV):
            for li in range(n_lanes):
                send_buf, recv_buf, send_sems, recv_sems, credit = \
                    lanes[li]
                if s == 0:
                    acc = ps[li]
                else:
                    prev[li].wait_recv()
                    acc = ps[li] + recv_buf[(s - 1) % 2, :, :].astype(
                        jnp.float32)
                    if s <= N_DEV - 3:
                        pl.semaphore_signal(
                            credit, inc=1, device_id=(ups[li],),
                            device_id_type=pl.DeviceIdType.MESH)
                if s < N_DEV - 1:
                    if s >= 2:
                        prev2[li].wait_send()
                    send_buf[s % 2, :, :] = acc.astype(WIRE_DTYPE)
                    if s >= 2:
                        pl.semaphore_wait(credit, 1)
                    rdma = pltpu.make_async_remote_copy(
                        src_ref=send_buf.at[s % 2],
                        dst_ref=recv_buf.at[s % 2],
                        send_sem=send_sems.at[s % 2],
                        recv_sem=recv_sems.at[s % 2],
                        device_id=(dst[li],),
                        device_id_type=pl.DeviceIdType.MESH,
                    )
                    rdma.start()
                    prev2[li] = prev[li]
                    prev[li] = rdma
                else:
                    out_ref[:, li * nq:(li + 1) * nq] = acc * scale
            if s < N_DEV - 1:
                ps = partials(s + 1)
        for li in range(n_lanes):
            prev2[li].wait_send()
            prev[li].wait_send()

    lane_scratch = []
    for _ in range(n_lanes):
        lane_scratch += [
            pltpu.VMEM((2, m_per, nq), WIRE_DTYPE),
            pltpu.VMEM((2, m_per, nq), WIRE_DTYPE),
            pltpu.SemaphoreType.DMA((2,)),
            pltpu.SemaphoreType.DMA((2,)),
            pltpu.SemaphoreType.REGULAR,
        ]

    return pl.pallas_call(
        body,
        out_shape=jax.ShapeDtypeStruct((m_per, n), jnp.float32),
        in_specs=[pl.BlockSpec(memory_space=pltpu.VMEM)] * 4
        + [pl.BlockSpec(memory_space=pltpu.SMEM)] * 4,
        out_specs=pl.BlockSpec(memory_space=pltpu.VMEM),
        scratch_shapes=lane_scratch,
        compiler_params=pltpu.CompilerParams(collective_id=0),
    )(x, w_mat, scale_x, scale_w, left, right, cs_cw, cs_ccw)


# device time: 113139 ns/iter; 3.7597x vs baseline; 1.0290x over previous
import jax
import jax.numpy as jnp
import numpy as np
from jax import lax
from jax.experimental import pallas as pl
from jax.experimental.pallas import tpu as pltpu

N_DEV = 32
N_SEG = 4
WIRE_DTYPE = jnp.bfloat16

_XY_TO_P = {(0, 0): 0, (1, 0): 1, (1, 1): 2, (0, 1): 3,
            (0, 2): 4, (1, 2): 5, (1, 3): 6, (0, 3): 7}


def _build_cycle() -> np.ndarray:
    path = []
    for y in range(4):
        zs = range(4) if y % 2 == 0 else range(3, -1, -1)
        path.extend((y, z) for z in zs)
    cycle = [(0, y, z) for (y, z) in path]
    cycle += [(1, y, z) for (y, z) in reversed(path)]
    return np.array([z * 8 + _XY_TO_P[(x, y)] for (x, y, z) in cycle],
                    dtype=np.int32)


PERM = _build_cycle()
PERM_INV = np.zeros(N_DEV, np.int32)
PERM_INV[PERM] = np.arange(N_DEV, dtype=np.int32)


def kernel(x, w_mat, scale_x, scale_w):
    k_global, _ = x.shape
    _, n = w_mat.shape
    m_per = k_global // N_DEV
    nh = n // 2
    nq = nh // N_SEG
    n_lanes = 2 * N_SEG

    perm = jnp.asarray(PERM)
    pos = jnp.asarray(PERM_INV)[lax.axis_index("i")]
    left = perm[(pos - 1) % N_DEV].reshape(1)
    right = perm[(pos + 1) % N_DEV].reshape(1)
    sidx = jnp.arange(N_DEV, dtype=jnp.int32)
    cs_cw = perm[(pos - 1 - sidx) % N_DEV]
    cs_ccw = perm[(pos + 1 + sidx) % N_DEV]

    def body(x_ref, w_ref, sx_ref, sw_ref, left_ref, right_ref,
             cs_cw_ref, cs_ccw_ref, out_ref, *scratch):
        lanes = [scratch[5 * li:5 * li + 5] for li in range(n_lanes)]

        lft = left_ref[0]
        rgt = right_ref[0]
        dst = [rgt] * N_SEG + [lft] * N_SEG
        ups = [lft] * N_SEG + [rgt] * N_SEG

        barrier_sem = pltpu.get_barrier_semaphore()
        for nbr in (lft, rgt):
            pl.semaphore_signal(barrier_sem, inc=1, device_id=(nbr,),
                                device_id_type=pl.DeviceIdType.MESH)
        pl.semaphore_wait(barrier_sem, 2)

        wl = w_ref[:, :nh].astype(jnp.bfloat16)
        wr = w_ref[:, nh:].astype(jnp.bfloat16)
        scale = sx_ref[0] * sw_ref[0]

        def partials(s):
            c1 = cs_cw_ref[s]
            c2 = cs_ccw_ref[s]
            x1 = x_ref[pl.ds(c1 * m_per, m_per), :].astype(jnp.bfloat16)
            x2 = x_ref[pl.ds(c2 * m_per, m_per), :].astype(jnp.bfloat16)
            dims = (((1,), (0,)), ((), ()))
            p_cw = lax.dot_general(x1, wl, dims,
                                   preferred_element_type=jnp.float32)
            p_ccw = lax.dot_general(x2, wr, dims,
                                    preferred_element_type=jnp.float32)
            ps = [p_cw[:, g * nq:(g + 1) * nq] for g in range(N_SEG)]
            ps += [p_ccw[:, g * nq:(g + 1) * nq] for g in range(N_SEG)]
            return ps

        ps = partials(0)
        prev = [None] * n_lanes
        prev2 = [None] * n_lanes
        for s in range(N_DEV):
            for li in range(n_lanes):
                send_buf, recv_buf, send_sems, recv_sems, credit = \
                    lanes[li]
                if s == 0:
                    acc = ps[li]
                else:
                    prev[li].wait_recv()
                    acc = ps[li] + recv_buf[(s - 1) % 2, :, :].astype(
                        jnp.float32)
                    if s <= N_DEV - 3:
                        pl.semaphore_signal(
                            credit, inc=1, device_id=(ups[li],),
                            device_id_type=pl.DeviceIdType.MESH)
                if s < N_DEV - 1:
                    if s >= 2:
                        prev2[li].wait_send()
                    send_buf[s % 2, :, :] = acc.astype(WIRE_DTYPE)
                    if s >= 2:
                        pl.semaphore_wait(credit, 1)
                    rdma = pltpu.make_async_remote_copy(
                        src_ref=send_buf.at[s % 2],
                        dst_ref=recv_buf.at[s % 2],
                        send_sem=send_sems.at[s % 2],
                        recv_sem=recv_sems.at[s % 2],
                        device_id=(dst[li],),
                        device_id_type=pl.DeviceIdType.MESH,
                    )
                    rdma.start()
                    prev2[li] = prev[li]
                    prev[li] = rdma
                else:
                    out_ref[:, li * nq:(li + 1) * nq] = acc * scale
            if s < N_DEV - 1:
                ps = partials(s + 1)
        for li in range(n_lanes):
            prev2[li].wait_send()
            prev[li].wait_send()

    lane_scratch = []
    for _ in range(n_lanes):
        lane_scratch += [
            pltpu.VMEM((2, m_per, nq), WIRE_DTYPE),
            pltpu.VMEM((2, m_per, nq), WIRE_DTYPE),
            pltpu.SemaphoreType.DMA((2,)),
            pltpu.SemaphoreType.DMA((2,)),
            pltpu.SemaphoreType.REGULAR,
        ]

    return pl.pallas_call(
        body,
        out_shape=jax.ShapeDtypeStruct((m_per, n), jnp.float32),
        in_specs=[pl.BlockSpec(memory_space=pltpu.VMEM)] * 4
        + [pl.BlockSpec(memory_space=pltpu.SMEM)] * 4,
        out_specs=pl.BlockSpec(memory_space=pltpu.VMEM),
        scratch_shapes=lane_scratch,
        compiler_params=pltpu.CompilerParams(collective_id=0),
    )(x, w_mat, scale_x, scale_w, left, right, cs_cw, cs_ccw)
